# baseline (speedup 1.0000x reference)
import jax
import jax.numpy as jnp
from jax import lax
from jax.experimental import pallas as pl
from jax.experimental.pallas import tpu as pltpu
from jax.experimental.pallas import tpu_sc as plsc

_WIN = 128
_WINS_PER_STEP = 2
_LANES = 16


def kernel(x, table):
    batch, seq = x.shape
    vocab, dim = table.shape
    bstep = _WIN * _WINS_PER_STEP
    x_t = x.T
    tab_lin = table.reshape(-1).reshape(vocab, dim)

    mesh = plsc.VectorSubcoreMesh(core_axis_name="core",
                                  subcore_axis_name="subcore")

    @pl.kernel(
        out_type=jax.ShapeDtypeStruct((seq, dim, batch), table.dtype),
        mesh=mesh,
        scratch_types=[
            pltpu.VMEM((_WINS_PER_STEP, _WIN, dim), jnp.float32),
            pltpu.SemaphoreType.DMA,
        ],
        compiler_params=pltpu.CompilerParams(use_tc_tiling_on_sc=False,
                                             needs_layout_passes=False),
    )
    def gather_kernel(tab_hbm, xt_hbm, out_hbm, rows_v, sem):
        iota = lax.iota(jnp.int32, _LANES)

        def body(idx_vmem, out_vmem):
            copies = [
                pltpu.async_copy(
                    tab_hbm.at[idx_vmem.at[0, pl.ds(w * _WIN, _WIN)]],
                    rows_v.at[w],
                    sem,
                )
                for w in range(_WINS_PER_STEP)
            ]
            for w in range(_WINS_PER_STEP):
                copies[w].wait()
                buf = rows_v.at[w]

                @pl.loop(0, dim)
                def _(d):
                    didx = jnp.full((_LANES,), 0, jnp.int32) + d
                    for c in range(_WIN // _LANES):
                        vals = plsc.load_gather(
                            buf, [c * _LANES + iota, didx])
                        out_vmem[0, d,
                                 pl.ds(w * _WIN + c * _LANES, _LANES)] = vals

        pltpu.emit_pipeline(
            body,
            grid=(seq, batch // bstep),
            in_specs=[
                pl.BlockSpec((1, bstep), index_map=lambda l, b: (l, b)),
            ],
            out_specs=[
                pl.BlockSpec((1, dim, bstep),
                             index_map=lambda l, b: (l, 0, b)),
            ],
            core_axis_name=("core", "subcore"),
            dimension_semantics=(pltpu.PARALLEL, pltpu.PARALLEL),
        )(xt_hbm, out_hbm)

    out_t = gather_kernel(tab_lin, x_t)
    return out_t.transpose(2, 0, 1)


# speedup vs baseline: 2.2747x; 2.2747x over previous
import jax
import jax.numpy as jnp
from jax import lax
from jax.experimental import pallas as pl
from jax.experimental.pallas import tpu as pltpu
from jax.experimental.pallas import tpu_sc as plsc

_WIN = 128
_NBUF = 5


def kernel(x, table):
    batch, seq = x.shape
    vocab, dim = table.shape
    pdim = 2 * dim
    tab128 = jnp.pad(table, ((0, 0), (0, pdim - dim)))
    x_t = x.T

    mesh = plsc.VectorSubcoreMesh(core_axis_name="core",
                                  subcore_axis_name="subcore")
    n_workers = 2 * 16
    assert batch % (n_workers * _WIN) == 0
    chunks_per_worker = batch // (n_workers * _WIN)

    @pl.kernel(
        out_type=jax.ShapeDtypeStruct((batch, seq, pdim), table.dtype),
        mesh=mesh,
        scratch_types=[
            pltpu.VMEM((seq, _WIN), jnp.int32),
            pltpu.VMEM((_NBUF * _WIN, pdim), jnp.float32),
            pltpu.SemaphoreType.DMA,
            pltpu.SemaphoreType.DMA,
            pltpu.SemaphoreType.DMA,
        ],
        compiler_params=pltpu.CompilerParams(needs_layout_passes=False),
    )
    def gather_kernel(tab_hbm, xt_hbm, out_hbm, idx_v, rows_v,
                      gsem, osem, isem):
        wid = lax.axis_index("subcore") * 2 + lax.axis_index("core")

        @pl.loop(0, chunks_per_worker)
        def _(chunk):
            b0 = (wid * chunks_per_worker + chunk) * _WIN
            pltpu.async_copy(xt_hbm.at[:, pl.ds(b0, _WIN)], idx_v,
                             isem).wait()

            for j in range(_NBUF - 1):
                pltpu.async_copy(tab_hbm.at[idx_v.at[j]],
                                 rows_v.at[pl.ds(j * _WIN, _WIN)], gsem)

            @pl.loop(0, seq // _NBUF)
            def _(i):
                l0 = i * _NBUF
                for nb in range(_NBUF):
                    l = l0 + nb
                    m = (nb + _NBUF - 1) % _NBUF
                    @pl.when(l >= 1)
                    def _():
                        pltpu.make_async_copy(
                            rows_v.at[pl.ds(m * _WIN, _WIN)],
                            out_hbm.at[pl.ds(b0, _WIN), l - 1, :],
                            osem).wait()

                    @pl.when(l + _NBUF - 1 < seq)
                    def _():
                        pltpu.async_copy(
                            tab_hbm.at[idx_v.at[l + _NBUF - 1]],
                            rows_v.at[pl.ds(m * _WIN, _WIN)], gsem)

                    pltpu.make_async_copy(
                        tab_hbm.at[idx_v.at[l]],
                        rows_v.at[pl.ds(nb * _WIN, _WIN)], gsem).wait()
                    pltpu.async_copy(
                        rows_v.at[pl.ds(nb * _WIN, _WIN)],
                        out_hbm.at[pl.ds(b0, _WIN), l, :], osem)

            pltpu.make_async_copy(
                rows_v.at[pl.ds(((seq - 1) % _NBUF) * _WIN, _WIN)],
                out_hbm.at[pl.ds(b0, _WIN), seq - 1, :], osem).wait()

    out3 = gather_kernel(tab128, x_t)
    return out3[:, :, :dim]
